# Initial kernel scaffold; baseline (speedup 1.0000x reference)
import jax
import jax.numpy as jnp
from jax.experimental import pallas as pl


def kernel(h_in, edge_index, edge_attr, weights):
    raise NotImplementedError("write your pallas kernel here")


# speedup vs baseline: 4.5850x; 4.5850x over previous
import functools

import jax
import jax.numpy as jnp
from jax import lax
from jax.experimental import pallas as pl
from jax.experimental.pallas import tpu as pltpu
from jax.experimental.pallas import tpu_sc as plsc

_N = 10000
_E = 320000
_DE = 16
_DIN = 128
_H = 128

_CHUNK = 128
_NSTREAMS = _E // _CHUNK
_NC = 2
_NS = 16
_NW = _NC * _NS
_ROWS_PER_TILE = 632
_NPAD = _ROWS_PER_TILE * _NS


def _sc_segment_sum(dst, edge_attr):
    mesh = plsc.VectorSubcoreMesh(core_axis_name="c", subcore_axis_name="s")

    @functools.partial(
        pl.kernel,
        mesh=mesh,
        compiler_params=pltpu.CompilerParams(use_tc_tiling_on_sc=False),
        out_type=[
            jax.ShapeDtypeStruct((_NC, _NPAD, _DE), jnp.float32),
            jax.ShapeDtypeStruct((_NC, _NPAD, _DE), jnp.float32),
        ],
        scratch_types=[
            pltpu.VMEM((1, _CHUNK), jnp.int32),
            pltpu.VMEM((_CHUNK, _DE), jnp.float32),
            pltpu.VMEM((_CHUNK, _DE), jnp.float32),
            pltpu.VMEM((_ROWS_PER_TILE, _DE), jnp.float32),
            pltpu.VMEM_SHARED((_NPAD, _DE), jnp.float32),
            pltpu.VMEM_SHARED((_NPAD, _DE), jnp.float32),
        ],
    )
    def seg(dst_hbm, attr_hbm, sums_hbm, cnts_hbm,
            idx_v, attr_v, ones_v, stage_v, acc_sum, acc_cnt):
        cid = lax.axis_index("c")
        sid = lax.axis_index("s")
        wid = sid * _NC + cid

        def init_body(i, carry):
            ones_v[i, :] = jnp.ones((_DE,), jnp.float32)
            return carry
        lax.fori_loop(0, _CHUNK, init_body, None)

        def zero_body(i, carry):
            stage_v[i, :] = jnp.zeros((_DE,), jnp.float32)
            return carry
        lax.fori_loop(0, _ROWS_PER_TILE, zero_body, None)

        row0 = sid * _ROWS_PER_TILE
        pltpu.sync_copy(stage_v, acc_sum.at[pl.ds(row0, _ROWS_PER_TILE)])
        pltpu.sync_copy(stage_v, acc_cnt.at[pl.ds(row0, _ROWS_PER_TILE)])
        plsc.subcore_barrier()

        ntrips = jnp.where(wid < _NSTREAMS % _NW,
                           _NSTREAMS // _NW + 1, _NSTREAMS // _NW)

        def body(t, carry):
            base = (wid + t * _NW) * _CHUNK
            pltpu.sync_copy(dst_hbm.at[pl.ds(base, _CHUNK)], idx_v.at[0])
            pltpu.sync_copy(attr_hbm.at[pl.ds(base, _CHUNK)], attr_v)
            pltpu.sync_copy(attr_v, acc_sum.at[idx_v.at[0]], add=True)
            pltpu.sync_copy(ones_v, acc_cnt.at[idx_v.at[0]], add=True)
            return carry
        lax.fori_loop(0, ntrips, body, None)

        plsc.subcore_barrier()

        pltpu.sync_copy(acc_sum.at[pl.ds(row0, _ROWS_PER_TILE)], stage_v)
        pltpu.sync_copy(stage_v, sums_hbm.at[cid, pl.ds(row0, _ROWS_PER_TILE)])
        pltpu.sync_copy(acc_cnt.at[pl.ds(row0, _ROWS_PER_TILE)], stage_v)
        pltpu.sync_copy(stage_v, cnts_hbm.at[cid, pl.ds(row0, _ROWS_PER_TILE)])

    return seg(dst, edge_attr)


_BLK = 1000


def _tc_body(h_ref, w1_ref, w2_ref, s_ref, c_ref, o_ref):
    s = s_ref[0] + s_ref[1]
    c = c_ref[0] + c_ref[1]
    mean = s / jnp.maximum(c, 1.0)
    o_ref[...] = (
        jnp.dot(h_ref[...], w1_ref[...],
                preferred_element_type=jnp.float32,
                precision=lax.Precision.HIGHEST)
        + jnp.dot(mean, w2_ref[...],
                  preferred_element_type=jnp.float32,
                  precision=lax.Precision.HIGHEST)
    )


def _tc_combine(h_in, w1t, w2t, sums, cnts):
    return pl.pallas_call(
        _tc_body,
        grid=(_N // _BLK,),
        in_specs=[
            pl.BlockSpec((_BLK, _DIN), lambda i: (i, 0)),
            pl.BlockSpec((_DIN, _H), lambda i: (0, 0)),
            pl.BlockSpec((_DE, _H), lambda i: (0, 0)),
            pl.BlockSpec((_NC, _BLK, _DE), lambda i: (0, i, 0)),
            pl.BlockSpec((_NC, _BLK, _DE), lambda i: (0, i, 0)),
        ],
        out_specs=pl.BlockSpec((_BLK, _H), lambda i: (i, 0)),
        out_shape=jax.ShapeDtypeStruct((_N, _H), jnp.float32),
    )(h_in, w1t, w2t, sums, cnts)


def kernel(h_in, edge_index, edge_attr, weights):
    dst = edge_index[1]
    sums, cnts = _sc_segment_sum(dst, edge_attr)
    w1t = weights[:, :_DIN].T
    w2t = weights[:, _DIN:].T
    return _tc_combine(h_in, w1t, w2t, sums, cnts)


# speedup vs baseline: 6.1031x; 1.3311x over previous
import functools

import jax
import jax.numpy as jnp
from jax import lax
from jax.experimental import pallas as pl
from jax.experimental.pallas import tpu as pltpu
from jax.experimental.pallas import tpu_sc as plsc

_N = 10000
_E = 320000
_DE = 16
_DIN = 128
_H = 128

_CHUNK = 128
_NSTREAMS = _E // _CHUNK
_NC = 2
_NS = 16
_NW = _NC * _NS
_ROWS_PER_TILE = 632
_NPAD = _ROWS_PER_TILE * _NS


def _sc_segment_sum(dst, edge_attr):
    mesh = plsc.VectorSubcoreMesh(core_axis_name="c", subcore_axis_name="s")

    @functools.partial(
        pl.kernel,
        mesh=mesh,
        compiler_params=pltpu.CompilerParams(use_tc_tiling_on_sc=False),
        out_type=[
            jax.ShapeDtypeStruct((_NC, _NPAD, _DE), jnp.float32),
            jax.ShapeDtypeStruct((_NC, _NPAD, _DE), jnp.float32),
        ],
        scratch_types=[
            pltpu.VMEM((2, 1, _CHUNK), jnp.int32),
            pltpu.VMEM((2, _CHUNK, _DE), jnp.float32),
            pltpu.VMEM((_CHUNK, _DE), jnp.float32),
            pltpu.VMEM((_ROWS_PER_TILE, _DE), jnp.float32),
            pltpu.VMEM_SHARED((_NPAD, _DE), jnp.float32),
            pltpu.VMEM_SHARED((_NPAD, _DE), jnp.float32),
            pltpu.SemaphoreType.DMA((2,)),
            pltpu.SemaphoreType.DMA((2,)),
        ],
    )
    def seg(dst_hbm, attr_hbm, sums_hbm, cnts_hbm,
            idx_v, attr_v, ones_v, stage_v, acc_sum, acc_cnt,
            isem, asem):
        cid = lax.axis_index("c")
        sid = lax.axis_index("s")
        wid = sid * _NC + cid

        def init_body(i, carry):
            ones_v[i, :] = jnp.ones((_DE,), jnp.float32)
            return carry
        lax.fori_loop(0, _CHUNK, init_body, None)

        def zero_body(i, carry):
            stage_v[i, :] = jnp.zeros((_DE,), jnp.float32)
            return carry
        lax.fori_loop(0, _ROWS_PER_TILE, zero_body, None)

        row0 = sid * _ROWS_PER_TILE
        pltpu.sync_copy(stage_v, acc_sum.at[pl.ds(row0, _ROWS_PER_TILE)])
        pltpu.sync_copy(stage_v, acc_cnt.at[pl.ds(row0, _ROWS_PER_TILE)])
        plsc.subcore_barrier()

        my_ntrips = jnp.where(wid < _NSTREAMS % _NW,
                              _NSTREAMS // _NW + 1, _NSTREAMS // _NW)

        def loads(t, b):
            base = (wid + t * _NW) * _CHUNK
            i_cp = pltpu.make_async_copy(
                dst_hbm.at[pl.ds(base, _CHUNK)], idx_v.at[b, 0], isem.at[b])
            a_cp = pltpu.make_async_copy(
                attr_hbm.at[pl.ds(base, _CHUNK)], attr_v.at[b], asem.at[b])
            return i_cp, a_cp

        def fire(t, b):
            i_cp, a_cp = loads(t, b)
            i_cp.start()
            a_cp.start()

        def consume(t, b):
            i_cp, a_cp = loads(t, b)
            i_cp.wait()
            a_cp.wait()
            pltpu.sync_copy(attr_v.at[b], acc_sum.at[idx_v.at[b, 0]], add=True)
            pltpu.sync_copy(ones_v, acc_cnt.at[idx_v.at[b, 0]], add=True)

        fire(0, 0)
        fire(1, 1)

        def body(i, carry):
            for b in range(2):
                t = 2 * i + b
                consume(t, b)

                @pl.when(t + 2 < my_ntrips)
                def _():
                    fire(t + 2, b)
            return carry
        lax.fori_loop(0, (_NSTREAMS // _NW) // 2, body, None)

        @pl.when(wid < _NSTREAMS % _NW)
        def _():
            consume(_NSTREAMS // _NW, 0)

        plsc.subcore_barrier()

        pltpu.sync_copy(acc_sum.at[pl.ds(row0, _ROWS_PER_TILE)], stage_v)
        pltpu.sync_copy(stage_v, sums_hbm.at[cid, pl.ds(row0, _ROWS_PER_TILE)])
        pltpu.sync_copy(acc_cnt.at[pl.ds(row0, _ROWS_PER_TILE)], stage_v)
        pltpu.sync_copy(stage_v, cnts_hbm.at[cid, pl.ds(row0, _ROWS_PER_TILE)])

    return seg(dst, edge_attr)


_BLK = 1000


def _tc_body(h_ref, w1_ref, w2_ref, s_ref, c_ref, o_ref):
    s = s_ref[0] + s_ref[1]
    c = c_ref[0] + c_ref[1]
    mean = s / jnp.maximum(c, 1.0)
    o_ref[...] = (
        jnp.dot(h_ref[...], w1_ref[...],
                preferred_element_type=jnp.float32,
                precision=lax.Precision.HIGHEST)
        + jnp.dot(mean, w2_ref[...],
                  preferred_element_type=jnp.float32,
                  precision=lax.Precision.HIGHEST)
    )


def _tc_combine(h_in, w1t, w2t, sums, cnts):
    return pl.pallas_call(
        _tc_body,
        grid=(_N // _BLK,),
        in_specs=[
            pl.BlockSpec((_BLK, _DIN), lambda i: (i, 0)),
            pl.BlockSpec((_DIN, _H), lambda i: (0, 0)),
            pl.BlockSpec((_DE, _H), lambda i: (0, 0)),
            pl.BlockSpec((_NC, _BLK, _DE), lambda i: (0, i, 0)),
            pl.BlockSpec((_NC, _BLK, _DE), lambda i: (0, i, 0)),
        ],
        out_specs=pl.BlockSpec((_BLK, _H), lambda i: (i, 0)),
        out_shape=jax.ShapeDtypeStruct((_N, _H), jnp.float32),
    )(h_in, w1t, w2t, sums, cnts)


def kernel(h_in, edge_index, edge_attr, weights):
    dst = edge_index[1]
    sums, cnts = _sc_segment_sum(dst, edge_attr)
    w1t = weights[:, :_DIN].T
    w2t = weights[:, _DIN:].T
    return _tc_combine(h_in, w1t, w2t, sums, cnts)
